# baseline (speedup 1.0000x reference)
import functools

import jax
import jax.numpy as jnp
from jax import lax
from jax.experimental import pallas as pl
from jax.experimental.pallas import tpu as pltpu
from jax.experimental.pallas import tpu_sc as plsc

H = 16
HD = 64
VD = 64
SCALE = HD ** -0.5


def _dot(a, b, dims):
    return jax.lax.dot_general(a, b, (dims, ((), ())),
                               preferred_element_type=jnp.float32)


def _proj_kernel(x_ref, w_ref, q_ref, kv_ref, wg_ref):
    o = _dot(x_ref[...], w_ref[...], ((1,), (1,)))
    d = H * HD
    for h in range(H):
        q_ref[h] = o[:, h * HD:(h + 1) * HD]
        kv_ref[h, :, :HD] = o[:, d + h * HD:d + (h + 1) * HD]
        kv_ref[h, :, HD:] = o[:, 2 * d + h * VD:2 * d + (h + 1) * VD]
    kv_ref[H] = jnp.zeros_like(kv_ref[H])
    wg_ref[...] = o[:, 3 * d:]


def _head_kernel(q_ref, kv_ref, o1_ref, ix_ref):
    h = pl.program_id(0)
    q = q_ref[0]
    k = kv_ref[0, :, :HD]
    v = kv_ref[0, :, HD:]
    n = q.shape[0]

    p = _dot(q, k, ((1,), (1,)))

    m1 = jnp.max(p, axis=1, keepdims=True)
    e1 = jnp.exp((p - m1) * SCALE)
    s1 = jnp.sum(e1, axis=1, keepdims=True)
    o1_ref[0] = _dot(e1, v, ((1,), (0,))) / s1

    iota = jax.lax.broadcasted_iota(jnp.int32, p.shape, 1)
    idx = jnp.min(jnp.where(p == m1, iota, 2 * n), axis=1, keepdims=True)
    valid = m1 > 0.0
    ix = jnp.where(valid, h * n + idx, H * n)
    ix_ref[0] = ix.reshape(8, n // 8)


def _branch2_kernel(q_ref, kvm_ref, o2_ref):
    q = q_ref[0]
    km = kvm_ref[0, :, :HD]
    vm = kvm_ref[0, :, HD:]
    p2 = _dot(q, km, ((1,), (1,)))
    m2 = jnp.max(p2, axis=1, keepdims=True)
    e2 = jnp.exp((p2 - m2) * SCALE)
    s2 = jnp.sum(e2, axis=1, keepdims=True)
    o2_ref[0] = _dot(e2, vm, ((1,), (0,))) / s2


def _combine_kernel(o1_ref, o2_ref, wg_ref, wout_ref, bout_ref, out_ref):
    a1 = _dot(o1_ref[0], wout_ref[0], ((1,), (0,)))
    a2 = _dot(o2_ref[0], wout_ref[0], ((1,), (0,)))
    for h in range(1, H):
        a1 = a1 + _dot(o1_ref[h], wout_ref[h], ((1,), (0,)))
        a2 = a2 + _dot(o2_ref[h], wout_ref[h], ((1,), (0,)))
    g = jax.nn.sigmoid(wg_ref[...])
    out_ref[...] = g * (a1 - a2) + a2 + bout_ref[...]


def _sc_gather(table, ix):
    info = plsc.get_sparse_core_info()
    nw = info.num_cores * info.num_subcores
    btot = ix.shape[0]
    bw = btot // nw
    nchunk = 2
    bc = bw // nchunk
    d = table.shape[1]
    mesh = plsc.VectorSubcoreMesh(core_axis_name="c", subcore_axis_name="s")

    @functools.partial(
        pl.kernel,
        out_type=jax.ShapeDtypeStruct((btot, d), jnp.float32),
        mesh=mesh,
        scratch_types=[
            pltpu.VMEM((bc,), jnp.int32),
            pltpu.VMEM((bc, d), jnp.float32),
            pltpu.SemaphoreType.DMA,
        ],
    )
    def gather_kernel(table_hbm, ix_hbm, kvm_hbm, idx_v, rows_v, sem):
        wid = lax.axis_index("s") * info.num_cores + lax.axis_index("c")
        for c in range(nchunk):
            base = wid * bw + c * bc
            pltpu.sync_copy(ix_hbm.at[pl.ds(base, bc)], idx_v)
            pltpu.async_copy(table_hbm.at[idx_v], rows_v, sem).wait()
            pltpu.sync_copy(rows_v, kvm_hbm.at[pl.ds(base, bc)])

    return gather_kernel(table, ix)


@jax.jit
def kernel(x, Wq, Wk, Wv, Ww, Wout, bout):
    b, n, d_in = x.shape
    d_out = Wout.shape[0]
    x2 = x.reshape(n, d_in)

    wcat = jnp.concatenate([Wq, Wk, Wv, Ww], axis=0)
    nt = 8
    q3, kv2, wg = pl.pallas_call(
        _proj_kernel,
        grid=(nt,),
        in_specs=[
            pl.BlockSpec((n // nt, d_in), lambda i: (i, 0)),
            pl.BlockSpec((4 * d_in, d_in), lambda i: (0, 0)),
        ],
        out_specs=[
            pl.BlockSpec((H, n // nt, HD), lambda i: (0, i, 0)),
            pl.BlockSpec((H + 1, n // nt, HD + VD), lambda i: (0, i, 0)),
            pl.BlockSpec((n // nt, d_in), lambda i: (i, 0)),
        ],
        out_shape=[
            jax.ShapeDtypeStruct((H, n, HD), jnp.float32),
            jax.ShapeDtypeStruct((H + 1, n, HD + VD), jnp.float32),
            jax.ShapeDtypeStruct((n, d_in), jnp.float32),
        ],
    )(x2, wcat)

    o1h, ix3 = pl.pallas_call(
        _head_kernel,
        grid=(H,),
        in_specs=[
            pl.BlockSpec((1, n, HD), lambda h: (h, 0, 0)),
            pl.BlockSpec((1, n, HD + VD), lambda h: (h, 0, 0)),
        ],
        out_specs=[
            pl.BlockSpec((1, n, VD), lambda h: (h, 0, 0)),
            pl.BlockSpec((1, 8, n // 8), lambda h: (h, 0, 0)),
        ],
        out_shape=[
            jax.ShapeDtypeStruct((H, n, VD), jnp.float32),
            jax.ShapeDtypeStruct((H, 8, n // 8), jnp.int32),
        ],
    )(q3, kv2)

    table = kv2.reshape((H + 1) * n, HD + VD)
    kvm = _sc_gather(table, ix3.reshape(H * n)).reshape(H, n, HD + VD)

    o2h = pl.pallas_call(
        _branch2_kernel,
        grid=(H,),
        in_specs=[
            pl.BlockSpec((1, n, HD), lambda h: (h, 0, 0)),
            pl.BlockSpec((1, n, HD + VD), lambda h: (h, 0, 0)),
        ],
        out_specs=pl.BlockSpec((1, n, VD), lambda h: (h, 0, 0)),
        out_shape=jax.ShapeDtypeStruct((H, n, VD), jnp.float32),
    )(q3, kvm)

    wout3 = Wout.T.reshape(H, VD, d_out)
    out = pl.pallas_call(
        _combine_kernel,
        grid=(nt,),
        in_specs=[
            pl.BlockSpec((H, n // nt, VD), lambda i: (0, i, 0)),
            pl.BlockSpec((H, n // nt, VD), lambda i: (0, i, 0)),
            pl.BlockSpec((n // nt, d_out), lambda i: (i, 0)),
            pl.BlockSpec((H, VD, d_out), lambda i: (0, 0, 0)),
            pl.BlockSpec((d_out,), lambda i: (0,)),
        ],
        out_specs=pl.BlockSpec((n // nt, d_out), lambda i: (i, 0)),
        out_shape=jax.ShapeDtypeStruct((n, d_out), jnp.float32),
    )(o1h, o2h, wg, wout3, bout)

    return out.reshape(b, n, d_out)


# speedup vs baseline: 1.3743x; 1.3743x over previous
import functools

import jax
import jax.numpy as jnp
from jax import lax
from jax.experimental import pallas as pl
from jax.experimental.pallas import tpu as pltpu
from jax.experimental.pallas import tpu_sc as plsc

H = 16
HD = 64
VD = 64
SCALE = HD ** -0.5


def _dot(a, b, dims):
    return jax.lax.dot_general(a, b, (dims, ((), ())),
                               preferred_element_type=jnp.float32)


def _proj_kernel(x_ref, w_ref, q_ref, kv_ref, wg_ref):
    o = _dot(x_ref[...], w_ref[...], ((1,), (1,)))
    d = H * HD
    for h in range(H):
        q_ref[h] = o[:, h * HD:(h + 1) * HD]
        kv_ref[h, :, :HD] = o[:, d + h * HD:d + (h + 1) * HD]
        kv_ref[h, :, HD:] = o[:, 2 * d + h * VD:2 * d + (h + 1) * VD]
    kv_ref[H] = jnp.zeros_like(kv_ref[H])
    wg_ref[...] = o[:, 3 * d:]


QT = 4


def _head_kernel(q_ref, kv_ref, o1_ref, ix_ref):
    h = pl.program_id(0)
    k = kv_ref[0, :, :HD]
    v = kv_ref[0, :, HD:]
    n = k.shape[0]
    tn = n // QT
    rt = tn // (n // 8)
    for t in range(QT):
        q = q_ref[0, t * tn:(t + 1) * tn]

        p = _dot(q, k, ((1,), (1,)))

        m1 = jnp.max(p, axis=1, keepdims=True)
        e1 = jnp.exp((p - m1) * SCALE)
        s1 = jnp.sum(e1, axis=1, keepdims=True)
        o1_ref[0, t * tn:(t + 1) * tn] = _dot(e1, v, ((1,), (0,))) / s1

        iota = jax.lax.broadcasted_iota(jnp.int32, p.shape, 1)
        idx = jnp.min(jnp.where(p == m1, iota, 2 * n), axis=1, keepdims=True)
        valid = m1 > 0.0
        ix = jnp.where(valid, h * n + idx, H * n)
        ix_ref[0, t * rt:(t + 1) * rt] = ix.reshape(rt, n // 8)


def _branch2_kernel(q_ref, kvm_ref, o2_ref):
    km = kvm_ref[0, :, :HD]
    vm = kvm_ref[0, :, HD:]
    n = km.shape[0]
    tn = n // QT
    for t in range(QT):
        q = q_ref[0, t * tn:(t + 1) * tn]
        p2 = _dot(q, km, ((1,), (1,)))
        m2 = jnp.max(p2, axis=1, keepdims=True)
        e2 = jnp.exp((p2 - m2) * SCALE)
        s2 = jnp.sum(e2, axis=1, keepdims=True)
        o2_ref[0, t * tn:(t + 1) * tn] = _dot(e2, vm, ((1,), (0,))) / s2


def _combine_kernel(o1_ref, o2_ref, wg_ref, wout_ref, bout_ref, out_ref):
    a1 = _dot(o1_ref[0], wout_ref[0], ((1,), (0,)))
    a2 = _dot(o2_ref[0], wout_ref[0], ((1,), (0,)))
    for h in range(1, H):
        a1 = a1 + _dot(o1_ref[h], wout_ref[h], ((1,), (0,)))
        a2 = a2 + _dot(o2_ref[h], wout_ref[h], ((1,), (0,)))
    g = jax.nn.sigmoid(wg_ref[...])
    out_ref[...] = g * (a1 - a2) + a2 + bout_ref[...]


def _sc_gather(table, ix):
    info = plsc.get_sparse_core_info()
    nw = info.num_cores * info.num_subcores
    btot = ix.shape[0]
    bw = btot // nw
    nchunk = 2
    bc = bw // nchunk
    d = table.shape[1]
    mesh = plsc.VectorSubcoreMesh(core_axis_name="c", subcore_axis_name="s")

    @functools.partial(
        pl.kernel,
        out_type=jax.ShapeDtypeStruct((btot, d), jnp.float32),
        mesh=mesh,
        scratch_types=[
            pltpu.VMEM((bc,), jnp.int32),
            pltpu.VMEM((bc, d), jnp.float32),
            pltpu.SemaphoreType.DMA,
        ],
    )
    def gather_kernel(table_hbm, ix_hbm, kvm_hbm, idx_v, rows_v, sem):
        wid = lax.axis_index("s") * info.num_cores + lax.axis_index("c")
        for c in range(nchunk):
            base = wid * bw + c * bc
            pltpu.sync_copy(ix_hbm.at[pl.ds(base, bc)], idx_v)
            pltpu.async_copy(table_hbm.at[idx_v], rows_v, sem).wait()
            pltpu.sync_copy(rows_v, kvm_hbm.at[pl.ds(base, bc)])

    return gather_kernel(table, ix)


@jax.jit
def kernel(x, Wq, Wk, Wv, Ww, Wout, bout):
    b, n, d_in = x.shape
    d_out = Wout.shape[0]
    x2 = x.reshape(n, d_in)

    wcat = jnp.concatenate([Wq, Wk, Wv, Ww], axis=0)
    nt = 8
    q3, kv2, wg = pl.pallas_call(
        _proj_kernel,
        grid=(nt,),
        in_specs=[
            pl.BlockSpec((n // nt, d_in), lambda i: (i, 0)),
            pl.BlockSpec((4 * d_in, d_in), lambda i: (0, 0)),
        ],
        out_specs=[
            pl.BlockSpec((H, n // nt, HD), lambda i: (0, i, 0)),
            pl.BlockSpec((H + 1, n // nt, HD + VD), lambda i: (0, i, 0)),
            pl.BlockSpec((n // nt, d_in), lambda i: (i, 0)),
        ],
        out_shape=[
            jax.ShapeDtypeStruct((H, n, HD), jnp.float32),
            jax.ShapeDtypeStruct((H + 1, n, HD + VD), jnp.float32),
            jax.ShapeDtypeStruct((n, d_in), jnp.float32),
        ],
    )(x2, wcat)

    o1h, ix3 = pl.pallas_call(
        _head_kernel,
        grid=(H,),
        in_specs=[
            pl.BlockSpec((1, n, HD), lambda h: (h, 0, 0)),
            pl.BlockSpec((1, n, HD + VD), lambda h: (h, 0, 0)),
        ],
        out_specs=[
            pl.BlockSpec((1, n, VD), lambda h: (h, 0, 0)),
            pl.BlockSpec((1, 8, n // 8), lambda h: (h, 0, 0)),
        ],
        out_shape=[
            jax.ShapeDtypeStruct((H, n, VD), jnp.float32),
            jax.ShapeDtypeStruct((H, 8, n // 8), jnp.int32),
        ],
    )(q3, kv2)

    table = kv2.reshape((H + 1) * n, HD + VD)
    kvm = _sc_gather(table, ix3.reshape(H * n)).reshape(H, n, HD + VD)

    o2h = pl.pallas_call(
        _branch2_kernel,
        grid=(H,),
        in_specs=[
            pl.BlockSpec((1, n, HD), lambda h: (h, 0, 0)),
            pl.BlockSpec((1, n, HD + VD), lambda h: (h, 0, 0)),
        ],
        out_specs=pl.BlockSpec((1, n, VD), lambda h: (h, 0, 0)),
        out_shape=jax.ShapeDtypeStruct((H, n, VD), jnp.float32),
    )(q3, kvm)

    wout3 = Wout.T.reshape(H, VD, d_out)
    out = pl.pallas_call(
        _combine_kernel,
        grid=(nt,),
        in_specs=[
            pl.BlockSpec((H, n // nt, VD), lambda i: (0, i, 0)),
            pl.BlockSpec((H, n // nt, VD), lambda i: (0, i, 0)),
            pl.BlockSpec((n // nt, d_out), lambda i: (i, 0)),
            pl.BlockSpec((H, VD, d_out), lambda i: (0, 0, 0)),
            pl.BlockSpec((d_out,), lambda i: (0,)),
        ],
        out_specs=pl.BlockSpec((n // nt, d_out), lambda i: (i, 0)),
        out_shape=jax.ShapeDtypeStruct((n, d_out), jnp.float32),
    )(o1h, o2h, wg, wout3, bout)

    return out.reshape(b, n, d_out)
